# baseline (speedup 1.0000x reference)
import functools

import jax
import jax.numpy as jnp
from jax import lax
from jax.experimental import pallas as pl
from jax.experimental.pallas import tpu as pltpu
from jax.experimental.pallas import tpu_sc as plsc

LANES = 16
NBUF = 3


def _sc_text_embed(text, table, freqs):
    B, NT = text.shape
    D = table.shape[1]
    info = plsc.get_sparse_core_info()
    NC, NS = info.num_cores, info.num_subcores
    NW = NC * NS
    rows_per_w = B // NW
    assert B % NW == 0 and D % LANES == 0

    NA = 112
    NB_REAL = NT - NA
    NB = ((NB_REAL + LANES - 1) // LANES) * LANES
    NTOK = rows_per_w * NT
    V = table.shape[0]
    VP = ((V + 7) // 8) * 8

    mesh = plsc.VectorSubcoreMesh(core_axis_name="c", subcore_axis_name="s")

    @functools.partial(
        pl.kernel,
        mesh=mesh,
        out_type=jax.ShapeDtypeStruct((B, NT, D), jnp.float32),
        scratch_types=[
            pltpu.VMEM((NTOK + LANES,), jnp.int32),
            pltpu.VMEM((NBUF, NA), jnp.int32),
            pltpu.VMEM((NBUF, NB), jnp.int32),
            pltpu.VMEM_SHARED((NT, D), jnp.float32),
            pltpu.VMEM((NBUF, NA, D), jnp.float32),
            pltpu.VMEM((NBUF, NB, D), jnp.float32),
            pltpu.VMEM_SHARED((VP, D), jnp.float32),
        ]
        + [pltpu.SemaphoreType.DMA] * (6 * NBUF),
    )
    def k(text_hbm, table_hbm, freqs_hbm, out_hbm,
          idx_all, idx_a, idx_b, freqs_v, rows_a, rows_b, table_sh, *sems):
        sem_ga = sems[0:NBUF]
        sem_gb = sems[NBUF:2 * NBUF]
        sem_oa = sems[2 * NBUF:3 * NBUF]
        sem_ob = sems[3 * NBUF:4 * NBUF]
        sem_fa = sems[4 * NBUF:5 * NBUF]
        sem_fb = sems[5 * NBUF:6 * NBUF]
        wid = lax.axis_index("s") * NC + lax.axis_index("c")
        base = wid * rows_per_w
        tok_base = base * NT

        idx_all[pl.ds(NTOK, LANES)] = jnp.zeros((LANES,), jnp.int32)
        pltpu.sync_copy(text_hbm.at[pl.ds(tok_base, NTOK)],
                        idx_all.at[pl.ds(0, NTOK)])

        @pl.when(lax.axis_index("s") == 0)
        def _():
            pltpu.sync_copy(table_hbm, table_sh)
            pltpu.sync_copy(freqs_hbm.at[pl.ds(0, NT)], freqs_v)
        plsc.subcore_barrier()

        def issue_prefill(s):
            pltpu.async_copy(freqs_v.at[pl.ds(0, NA)], rows_a.at[s],
                             sem_fa[s])
            pltpu.async_copy(freqs_v.at[pl.ds(NA, NB_REAL)],
                             rows_b.at[s, pl.ds(0, NB_REAL)], sem_fb[s])

        def wait_prefill(s):
            pltpu.make_async_copy(freqs_v.at[pl.ds(0, NA)], rows_a.at[s],
                                  sem_fa[s]).wait()
            pltpu.make_async_copy(freqs_v.at[pl.ds(NA, NB_REAL)],
                                  rows_b.at[s, pl.ds(0, NB_REAL)],
                                  sem_fb[s]).wait()

        def prep_idx(r, s):
            o = r * NT
            for i in range(NA // LANES):
                idx_a[s, pl.ds(i * LANES, LANES)] = (
                    idx_all[pl.ds(o + i * LANES, LANES)] + 1)
            for i in range(NB // LANES):
                idx_b[s, pl.ds(i * LANES, LANES)] = (
                    idx_all[pl.ds(o + NA + i * LANES, LANES)] + 1)

        def issue_gather_add(s):
            pltpu.async_copy(table_sh.at[idx_a.at[s]], rows_a.at[s],
                             sem_ga[s], add=True)
            pltpu.async_copy(table_sh.at[idx_b.at[s]], rows_b.at[s],
                             sem_gb[s], add=True)

        def wait_gather(s):
            pltpu.make_async_copy(table_sh.at[idx_a.at[s]], rows_a.at[s],
                                  sem_ga[s]).wait()
            pltpu.make_async_copy(table_sh.at[idx_b.at[s]], rows_b.at[s],
                                  sem_gb[s]).wait()

        def issue_out(r, s):
            b = base + r
            pltpu.async_copy(rows_a.at[s], out_hbm.at[b, pl.ds(0, NA)],
                             sem_oa[s])
            pltpu.async_copy(rows_b.at[s, pl.ds(0, NB_REAL)],
                             out_hbm.at[b, pl.ds(NA, NB_REAL)], sem_ob[s])

        def wait_out(r, s):
            b = base + r
            pltpu.make_async_copy(rows_a.at[s], out_hbm.at[b, pl.ds(0, NA)],
                                  sem_oa[s]).wait()
            pltpu.make_async_copy(rows_b.at[s, pl.ds(0, NB_REAL)],
                                  out_hbm.at[b, pl.ds(NA, NB_REAL)],
                                  sem_ob[s]).wait()

        def prep_gather(r, s):
            issue_prefill(s)
            prep_idx(r, s)
            wait_prefill(s)
            issue_gather_add(s)

        def process(r, s):
            wait_gather(s)
            issue_out(r, s)

        prep_gather(0, 0)

        def body(kk, c):
            r0 = kk * NBUF
            for d in range(NBUF):
                r = r0 + d
                sn = (d + 1) % NBUF
                if d < NBUF - 1:
                    @pl.when(kk > 0)
                    def _():
                        wait_out(r + 1 - NBUF, sn)
                else:
                    wait_out(r + 1 - NBUF, sn)
                prep_gather(r + 1, sn)
                process(r, d)
            return c

        n_main = (rows_per_w - 2) // NBUF
        assert n_main * NBUF == rows_per_w - 2
        lax.fori_loop(0, n_main, body, 0)

        r30, r31 = rows_per_w - 2, rows_per_w - 1
        wait_out(r30 - 2, (r30 - 2) % NBUF)
        prep_gather(r31, r31 % NBUF)
        process(r30, r30 % NBUF)
        process(r31, r31 % NBUF)
        wait_out(r30 - 1, (r30 - 1) % NBUF)
        wait_out(r30, r30 % NBUF)
        wait_out(r31, r31 % NBUF)

    table_p = jnp.concatenate(
        [table, jnp.zeros((VP - V, D), table.dtype)]) if VP != V else table
    return k(text.reshape(-1), table_p, freqs)


def kernel(text, text_embed_table, freqs_cis):
    return _sc_text_embed(text, text_embed_table, freqs_cis)


# speedup vs baseline: 1.2434x; 1.2434x over previous
import functools

import jax
import jax.numpy as jnp
from jax import lax
from jax.experimental import pallas as pl
from jax.experimental.pallas import tpu as pltpu
from jax.experimental.pallas import tpu_sc as plsc

LANES = 16
NBUF = 3
JBLK = 8


def _sc_text_embed(text, table, freqs):
    B, NT = text.shape
    D = table.shape[1]
    info = plsc.get_sparse_core_info()
    NC, NS = info.num_cores, info.num_subcores
    NW = NC * NS
    rows_per_w = B // NW
    assert B % NW == 0 and D % LANES == 0

    NA = 112
    NB_REAL = NT - NA
    NB = ((NB_REAL + LANES - 1) // LANES) * LANES
    NTOK = rows_per_w * NT
    assert NA % JBLK == 0 and NB_REAL % JBLK == 0
    V = table.shape[0]
    VP = ((V + 7) // 8) * 8

    mesh = plsc.VectorSubcoreMesh(core_axis_name="c", subcore_axis_name="s")

    @functools.partial(
        pl.kernel,
        mesh=mesh,
        out_type=jax.ShapeDtypeStruct((B, NT, D), jnp.float32),
        scratch_types=[
            pltpu.VMEM((NTOK + LANES,), jnp.int32),
            pltpu.VMEM((NBUF, NA), jnp.int32),
            pltpu.VMEM((NBUF, NB), jnp.int32),
            pltpu.VMEM((NT, D), jnp.float32),
            pltpu.VMEM((NBUF, NA, D), jnp.float32),
            pltpu.VMEM((NBUF, NB, D), jnp.float32),
            pltpu.VMEM_SHARED((VP, D), jnp.float32),
        ]
        + [pltpu.SemaphoreType.DMA] * (4 * NBUF),
    )
    def k(text_hbm, table_hbm, freqs_hbm, out_hbm,
          idx_all, idx_a, idx_b, freqs_v, rows_a, rows_b, table_sh, *sems):
        sem_ga = sems[0:NBUF]
        sem_gb = sems[NBUF:2 * NBUF]
        sem_oa = sems[2 * NBUF:3 * NBUF]
        sem_ob = sems[3 * NBUF:4 * NBUF]
        wid = lax.axis_index("s") * NC + lax.axis_index("c")
        base = wid * rows_per_w
        tok_base = base * NT

        pltpu.sync_copy(freqs_hbm.at[pl.ds(0, NT)], freqs_v)
        idx_all[pl.ds(NTOK, LANES)] = jnp.zeros((LANES,), jnp.int32)
        pltpu.sync_copy(text_hbm.at[pl.ds(tok_base, NTOK)],
                        idx_all.at[pl.ds(0, NTOK)])

        @pl.when(lax.axis_index("s") == 0)
        def _():
            pltpu.sync_copy(table_hbm, table_sh)
        plsc.subcore_barrier()

        def prep_gather(r, s):
            o = r * NT
            for i in range(NA // LANES):
                idx_a[s, pl.ds(i * LANES, LANES)] = (
                    idx_all[pl.ds(o + i * LANES, LANES)] + 1)
            for i in range(NB // LANES):
                idx_b[s, pl.ds(i * LANES, LANES)] = (
                    idx_all[pl.ds(o + NA + i * LANES, LANES)] + 1)
            pltpu.async_copy(table_sh.at[idx_a.at[s]], rows_a.at[s],
                             sem_ga[s])
            pltpu.async_copy(table_sh.at[idx_b.at[s]], rows_b.at[s],
                             sem_gb[s])

        def wait_gather(s):
            pltpu.make_async_copy(table_sh.at[idx_a.at[s]], rows_a.at[s],
                                  sem_ga[s]).wait()
            pltpu.make_async_copy(table_sh.at[idx_b.at[s]], rows_b.at[s],
                                  sem_gb[s]).wait()

        def issue_out(r, s):
            b = base + r
            pltpu.async_copy(rows_a.at[s], out_hbm.at[b, pl.ds(0, NA)],
                             sem_oa[s])
            pltpu.async_copy(rows_b.at[s, pl.ds(0, NB_REAL)],
                             out_hbm.at[b, pl.ds(NA, NB_REAL)], sem_ob[s])

        def wait_out(r, s):
            b = base + r
            pltpu.make_async_copy(rows_a.at[s], out_hbm.at[b, pl.ds(0, NA)],
                                  sem_oa[s]).wait()
            pltpu.make_async_copy(rows_b.at[s, pl.ds(0, NB_REAL)],
                                  out_hbm.at[b, pl.ds(NA, NB_REAL)],
                                  sem_ob[s]).wait()

        def add_freqs_a(s):
            def add_a(i, c):
                j8 = i * JBLK
                for jj in range(JBLK):
                    for ch in range(D // LANES):
                        sl = pl.ds(ch * LANES, LANES)
                        plsc.addupdate(rows_a.at[s, j8 + jj, sl],
                                       freqs_v[j8 + jj, sl])
                return c
            lax.fori_loop(0, NA // JBLK, add_a, 0)

        def add_freqs_b(s):
            def add_b(i, c):
                j8 = i * JBLK
                for jj in range(JBLK):
                    for ch in range(D // LANES):
                        sl = pl.ds(ch * LANES, LANES)
                        plsc.addupdate(rows_b.at[s, j8 + jj, sl],
                                       freqs_v[NA + j8 + jj, sl])
                return c
            lax.fori_loop(0, NB_REAL // JBLK, add_b, 0)

        def process(r, s):
            b = base + r
            pltpu.make_async_copy(table_sh.at[idx_a.at[s]], rows_a.at[s],
                                  sem_ga[s]).wait()
            add_freqs_a(s)
            pltpu.async_copy(rows_a.at[s], out_hbm.at[b, pl.ds(0, NA)],
                             sem_oa[s])
            pltpu.make_async_copy(table_sh.at[idx_b.at[s]], rows_b.at[s],
                                  sem_gb[s]).wait()
            add_freqs_b(s)
            pltpu.async_copy(rows_b.at[s, pl.ds(0, NB_REAL)],
                             out_hbm.at[b, pl.ds(NA, NB_REAL)], sem_ob[s])

        prep_gather(0, 0)

        def body(kk, c):
            r0 = kk * NBUF
            for d in range(NBUF):
                r = r0 + d
                sn = (d + 1) % NBUF
                if d < NBUF - 1:
                    @pl.when(kk > 0)
                    def _():
                        wait_out(r + 1 - NBUF, sn)
                else:
                    wait_out(r + 1 - NBUF, sn)
                prep_gather(r + 1, sn)
                process(r, d)
            return c

        n_main = (rows_per_w - 2) // NBUF
        assert n_main * NBUF == rows_per_w - 2
        lax.fori_loop(0, n_main, body, 0)

        r30, r31 = rows_per_w - 2, rows_per_w - 1
        wait_out(r30 - 2, (r30 - 2) % NBUF)
        prep_gather(r31, r31 % NBUF)
        process(r30, r30 % NBUF)
        process(r31, r31 % NBUF)
        wait_out(r30 - 1, (r30 - 1) % NBUF)
        wait_out(r30, r30 % NBUF)
        wait_out(r31, r31 % NBUF)

    table_p = jnp.concatenate(
        [table, jnp.zeros((VP - V, D), table.dtype)]) if VP != V else table
    return k(text.reshape(-1), table_p, freqs)


def kernel(text, text_embed_table, freqs_cis):
    return _sc_text_embed(text, text_embed_table, freqs_cis)
